# Initial kernel scaffold; baseline (speedup 1.0000x reference)
import jax
import jax.numpy as jnp
from jax.experimental import pallas as pl


def kernel(x_names, x_types, x_behaviors, edge_index, batch, name_table, type_table, W1, b1, W2, b2, Wc, bc):
    raise NotImplementedError("write your pallas kernel here")


# speedup vs baseline: 11.0049x; 11.0049x over previous
import functools

import jax
import jax.numpy as jnp
from jax import lax
from jax.experimental import pallas as pl
from jax.experimental.pallas import tpu as pltpu
from jax.experimental.pallas import tpu_sc as plsc

N = 50000
E = 800000
G = 64
HID = 64
HALF = 25000
NS = 16
NC = 2
SEG = 1568
DEG_PAD = NS * SEG
DUMMY = HALF
ZROWS = HALF + 8
EPT = E // NS
CHUNK = 2000
NCHUNK = EPT // CHUNK
BATCH = 80
NBATCH = CHUNK // BATCH
ROWB = 128
NROWB = (N + ROWB - 1) // ROWB
BLK = 200
NBLK = N // BLK


def _sc_mesh():
    return plsc.VectorSubcoreMesh(
        core_axis_name="c", subcore_axis_name="s", num_cores=NC, num_subcores=NS
    )


_SC_PARAMS = pltpu.CompilerParams(use_tc_tiling_on_sc=False)


def _k1_body(edge_dst, x_names, x_types, name_table, type_table,
             cnt_out, nf_out, tf_out,
             dst_st, idxbuf, ones_v, seg_v, nidx_v, nrows_v, trows_v,
             deg_sh, gsem, ssem):
    c = lax.axis_index("c")
    s = lax.axis_index("s")
    w = c * NS + s
    lo = c * HALF

    for v in range(SEG // 16):
        seg_v[pl.ds(v * 16, 16)] = jnp.zeros((16,), jnp.float32)
    pltpu.sync_copy(seg_v, deg_sh.at[pl.ds(s * SEG, SEG)])
    for v in range(BATCH // 16):
        ones_v[pl.ds(v * 16, 16)] = jnp.ones((16,), jnp.float32)
    plsc.subcore_barrier()

    def chunk_body(ch, carry):
        base = s * EPT + ch * CHUNK
        pltpu.sync_copy(edge_dst.at[pl.ds(base, CHUNK)], dst_st)
        for v in range(CHUNK // 16):
            d16 = dst_st[pl.ds(v * 16, 16)]
            inh = (d16 >= lo) & (d16 < lo + HALF)
            idx16 = jnp.where(inh, d16 - lo, DUMMY)
            idxbuf[v // (BATCH // 16), pl.ds((v % (BATCH // 16)) * 16, 16)] = idx16
        descs = [
            pltpu.async_copy(ones_v, deg_sh.at[idxbuf.at[b]], ssem, add=True)
            for b in range(NBATCH)
        ]
        for d in descs:
            d.wait()
        return carry

    lax.fori_loop(0, NCHUNK, chunk_body, 0)
    plsc.subcore_barrier()
    start = jnp.minimum(s * SEG, HALF - SEG)
    pltpu.sync_copy(deg_sh.at[pl.ds(start, SEG)], seg_v)
    pltpu.sync_copy(seg_v, cnt_out.at[pl.ds(lo + start, SEG)])

    def gbatch(j, carry):
        b = w + NC * NS * j

        @pl.when(b < NROWB)
        def _():
            st = jnp.minimum(b * ROWB, N - ROWB)
            pltpu.sync_copy(x_names.at[pl.ds(st, ROWB)], nidx_v)
            pltpu.async_copy(name_table.at[nidx_v], nrows_v, gsem).wait()
            pltpu.sync_copy(nrows_v, nf_out.at[pl.ds(st, ROWB)])
            pltpu.sync_copy(x_types.at[pl.ds(st, ROWB)], nidx_v)
            pltpu.async_copy(type_table.at[nidx_v], trows_v, gsem).wait()
            pltpu.sync_copy(trows_v, tf_out.at[pl.ds(st, ROWB)])

        return carry

    lax.fori_loop(0, (NROWB + NC * NS - 1) // (NC * NS), gbatch, 0)


def _k1(edge_dst, x_names, x_types, name_table, type_table):
    f = pl.kernel(
        _k1_body,
        out_type=[
            jax.ShapeDtypeStruct((N,), jnp.float32),
            jax.ShapeDtypeStruct((N, 64), jnp.float32),
            jax.ShapeDtypeStruct((N, 16), jnp.float32),
        ],
        mesh=_sc_mesh(),
        scratch_types=[
            pltpu.VMEM((CHUNK,), jnp.int32),
            pltpu.VMEM((NBATCH, BATCH), jnp.int32),
            pltpu.VMEM((BATCH,), jnp.float32),
            pltpu.VMEM((SEG,), jnp.float32),
            pltpu.VMEM((ROWB,), jnp.int32),
            pltpu.VMEM((ROWB, 64), jnp.float32),
            pltpu.VMEM((ROWB, 16), jnp.float32),
            pltpu.VMEM_SHARED((DEG_PAD,), jnp.float32),
            pltpu.SemaphoreType.DMA,
            pltpu.SemaphoreType.DMA,
        ],
        compiler_params=_SC_PARAMS,
    )
    return f(edge_dst, x_names, x_types, name_table, type_table)


SUBSEG = SEG // 8


def _k2_body(y, edge_src, edge_dst, z_out,
             src_st, dst_st, idxbuf, gbuf0, gbuf1, bounce, z_sh, gsem, ssem):
    c = lax.axis_index("c")
    s = lax.axis_index("s")
    lo = c * HALF
    start = jnp.minimum(s * SEG, HALF - SEG)
    gbufs = [gbuf0, gbuf1]

    for k in range(8):
        pltpu.sync_copy(y.at[pl.ds(lo + start + k * SUBSEG, SUBSEG)], bounce)
        pltpu.sync_copy(bounce, z_sh.at[pl.ds(start + k * SUBSEG, SUBSEG)])
    plsc.subcore_barrier()

    def chunk_body(ch, carry):
        base = s * EPT + ch * CHUNK
        pltpu.sync_copy(edge_src.at[pl.ds(base, CHUNK)], src_st)
        pltpu.sync_copy(edge_dst.at[pl.ds(base, CHUNK)], dst_st)
        for v in range(CHUNK // 16):
            d16 = dst_st[pl.ds(v * 16, 16)]
            inh = (d16 >= lo) & (d16 < lo + HALF)
            idx16 = jnp.where(inh, d16 - lo, DUMMY)
            idxbuf[v // (BATCH // 16), pl.ds((v % (BATCH // 16)) * 16, 16)] = idx16
        gd = [None] * NBATCH
        sd = [None] * NBATCH
        gd[0] = pltpu.async_copy(y.at[src_st.at[pl.ds(0, BATCH)]], gbufs[0], gsem)
        for b in range(NBATCH):
            if b >= 1:
                sd[b - 1].wait()
            if b + 1 < NBATCH:
                gd[b + 1] = pltpu.async_copy(
                    y.at[src_st.at[pl.ds((b + 1) * BATCH, BATCH)]],
                    gbufs[(b + 1) % 2], gsem)
            gd[b].wait()
            sd[b] = pltpu.async_copy(gbufs[b % 2], z_sh.at[idxbuf.at[b]], ssem,
                                     add=True)
        sd[NBATCH - 1].wait()
        return carry

    lax.fori_loop(0, NCHUNK, chunk_body, 0)
    plsc.subcore_barrier()
    for k in range(8):
        pltpu.sync_copy(z_sh.at[pl.ds(start + k * SUBSEG, SUBSEG)], bounce)
        pltpu.sync_copy(bounce, z_out.at[pl.ds(lo + start + k * SUBSEG, SUBSEG)])


def _k2(y, edge_src, edge_dst):
    f = pl.kernel(
        _k2_body,
        out_type=jax.ShapeDtypeStruct((N, HID), jnp.float32),
        mesh=_sc_mesh(),
        scratch_types=[
            pltpu.VMEM((CHUNK,), jnp.int32),
            pltpu.VMEM((CHUNK,), jnp.int32),
            pltpu.VMEM((NBATCH, BATCH), jnp.int32),
            pltpu.VMEM((BATCH, HID), jnp.float32),
            pltpu.VMEM((BATCH, HID), jnp.float32),
            pltpu.VMEM((SUBSEG, HID), jnp.float32),
            pltpu.VMEM_SHARED((ZROWS, HID), jnp.float32),
            pltpu.SemaphoreType.DMA,
            pltpu.SemaphoreType.DMA,
        ],
        compiler_params=_SC_PARAMS,
    )
    return f(y, edge_src, edge_dst)


def _tc1_body(nf, tf, bh, cnt, w1a, w1b, w1c, o):
    acc = jnp.dot(nf[...], w1a[...], preferred_element_type=jnp.float32)
    acc += jnp.dot(tf[...], w1b[...], preferred_element_type=jnp.float32)
    acc += jnp.dot(bh[...], w1c[...], preferred_element_type=jnp.float32)
    dinv = lax.rsqrt(cnt[...] + 1.0)
    o[...] = acc * dinv


def _tc1(nf, tf, bh, cnt2, w1a, w1b, w1c):
    return pl.pallas_call(
        _tc1_body,
        grid=(NBLK,),
        in_specs=[
            pl.BlockSpec((BLK, 64), lambda i: (i, 0)),
            pl.BlockSpec((BLK, 16), lambda i: (i, 0)),
            pl.BlockSpec((BLK, 48), lambda i: (i, 0)),
            pl.BlockSpec((BLK, 1), lambda i: (i, 0)),
            pl.BlockSpec((64, HID), lambda i: (0, 0)),
            pl.BlockSpec((16, HID), lambda i: (0, 0)),
            pl.BlockSpec((48, HID), lambda i: (0, 0)),
        ],
        out_specs=pl.BlockSpec((BLK, HID), lambda i: (i, 0)),
        out_shape=jax.ShapeDtypeStruct((N, HID), jnp.float32),
    )(nf, tf, bh, cnt2, w1a, w1b, w1c)


def _tc2_body(z, cnt, w2, b1r, o):
    dinv = lax.rsqrt(cnt[...] + 1.0)
    h = jnp.maximum(z[...] * dinv + b1r[...], 0.0)
    o[...] = jnp.dot(h, w2[...], preferred_element_type=jnp.float32) * dinv


def _tc2(z1, cnt2, W2, b1r):
    return pl.pallas_call(
        _tc2_body,
        grid=(NBLK,),
        in_specs=[
            pl.BlockSpec((BLK, HID), lambda i: (i, 0)),
            pl.BlockSpec((BLK, 1), lambda i: (i, 0)),
            pl.BlockSpec((HID, HID), lambda i: (0, 0)),
            pl.BlockSpec((1, HID), lambda i: (0, 0)),
        ],
        out_specs=pl.BlockSpec((BLK, HID), lambda i: (i, 0)),
        out_shape=jax.ShapeDtypeStruct((N, HID), jnp.float32),
    )(z1, cnt2, W2, b1r)


def _tc3_body(z, cnt, bat, b2r, wc, bcr, o, acc, gcnt):
    i = pl.program_id(0)

    @pl.when(i == 0)
    def _():
        acc[...] = jnp.zeros_like(acc)
        gcnt[...] = jnp.zeros_like(gcnt)

    dinv = lax.rsqrt(cnt[...] + 1.0)
    h = jnp.maximum(z[...] * dinv + b2r[...], 0.0)
    onehot = (bat[...] == lax.broadcasted_iota(jnp.int32, (BLK, G), 1)
              ).astype(jnp.float32)
    acc[...] += lax.dot_general(onehot, h, (((0,), (0,)), ((), ())),
                                preferred_element_type=jnp.float32)
    gcnt[...] += lax.dot_general(onehot, jnp.ones((BLK, 1), jnp.float32),
                                 (((0,), (0,)), ((), ())),
                                 preferred_element_type=jnp.float32)

    @pl.when(i == NBLK - 1)
    def _():
        pooled = acc[...] / jnp.maximum(gcnt[...], 1.0)
        o[...] = jnp.dot(pooled, wc[...], preferred_element_type=jnp.float32) \
            + bcr[...]


def _tc3(z2, cnt2, bat2, b2r, Wc, bcr):
    return pl.pallas_call(
        _tc3_body,
        grid=(NBLK,),
        in_specs=[
            pl.BlockSpec((BLK, HID), lambda i: (i, 0)),
            pl.BlockSpec((BLK, 1), lambda i: (i, 0)),
            pl.BlockSpec((BLK, 1), lambda i: (i, 0)),
            pl.BlockSpec((1, HID), lambda i: (0, 0)),
            pl.BlockSpec((HID, 2), lambda i: (0, 0)),
            pl.BlockSpec((1, 2), lambda i: (0, 0)),
        ],
        out_specs=pl.BlockSpec((G, 2), lambda i: (0, 0)),
        out_shape=jax.ShapeDtypeStruct((G, 2), jnp.float32),
        scratch_shapes=[
            pltpu.VMEM((G, HID), jnp.float32),
            pltpu.VMEM((G, 1), jnp.float32),
        ],
    )(z2, cnt2, bat2, b2r, Wc, bcr)


def kernel(x_names, x_types, x_behaviors, edge_index, batch,
           name_table, type_table, W1, b1, W2, b2, Wc, bc):
    xn = x_names.astype(jnp.int32)
    xt = x_types.astype(jnp.int32)
    bh = x_behaviors.astype(jnp.float32)
    esrc = edge_index[0].astype(jnp.int32)
    edst = edge_index[1].astype(jnp.int32)
    bat2 = batch.astype(jnp.int32).reshape(N, 1)
    nt = name_table.astype(jnp.float32)
    tt = type_table.astype(jnp.float32)
    cnt, nf, tf = _k1(edst, xn, xt, nt, tt)
    cnt2 = cnt.reshape(N, 1)

    W1f = W1.astype(jnp.float32)
    y1 = _tc1(nf, tf, bh, cnt2, W1f[:64], W1f[64:80], W1f[80:])
    z1 = _k2(y1, esrc, edst)
    y2 = _tc2(z1, cnt2, W2.astype(jnp.float32),
              b1.astype(jnp.float32).reshape(1, HID))
    z2 = _k2(y2, esrc, edst)
    return _tc3(z2, cnt2, bat2, b2.astype(jnp.float32).reshape(1, HID),
                Wc.astype(jnp.float32), bc.astype(jnp.float32).reshape(1, 2))
